# baseline (speedup 1.0000x reference)
import functools

import jax
import jax.numpy as jnp
from jax import lax
from jax.experimental import pallas as pl
from jax.experimental.pallas import tpu as pltpu
from jax.experimental.pallas import tpu_sc as plsc

N = 10000
E = 320000
NP = 10240
EP = 327680
CHUNK = 128
NCHUNKS = EP // CHUNK
NC, NS = 2, 16
NW = NC * NS
CPW = NCHUNKS // NW
TROWS = NP // NS
ZCH = 128

_MESH = dict(core_axis_name="c", subcore_axis_name="s", num_cores=NC,
             num_subcores=NS)


def _sc_scatter():
    @functools.partial(
        pl.kernel,
        out_type=jax.ShapeDtypeStruct((NC, NP, 128), jnp.float32),
        mesh=plsc.VectorSubcoreMesh(**_MESH),
        scratch_types=[
            pltpu.VMEM((4, 2, CHUNK), jnp.int32),
            pltpu.VMEM((2, CHUNK, 128), jnp.float32),
            pltpu.VMEM_SHARED((NP, 128), jnp.float32),
        ] + [pltpu.SemaphoreType.DMA] * 8,
    )
    def k(h_hbm, sd_hbm, zeros_hbm, out_hbm, ib, rows, acc,
          si0, si1, si2, si3, sg0, sg1, ss0, ss1):
        si = (si0, si1, si2, si3)
        sg = (sg0, sg1)
        ss = (ss0, ss1)
        c = lax.axis_index("c")
        t = lax.axis_index("s")
        wid = c * NS + t
        base = wid * CPW
        pltpu.sync_copy(zeros_hbm, rows.at[0].at[pl.ds(0, ZCH)])
        for z in range(TROWS // ZCH):
            pltpu.sync_copy(rows.at[0].at[pl.ds(0, ZCH)],
                            acc.at[pl.ds(t * TROWS + z * ZCH, ZCH)])
        pltpu.async_copy(sd_hbm.at[base], ib.at[0], si0)
        pltpu.async_copy(sd_hbm.at[base + 1], ib.at[1], si1)
        pltpu.make_async_copy(sd_hbm.at[0], ib.at[0], si0).wait()
        pltpu.async_copy(h_hbm.at[ib.at[0].at[0]], rows.at[0], sg0)
        plsc.subcore_barrier()

        def quarter(j, b, q):
            nb, q1, q2 = 1 - b, (q + 1) % 4, (q + 2) % 4
            pltpu.make_async_copy(h_hbm.at[pl.ds(0, CHUNK)], rows.at[b],
                                  sg[b]).wait()
            pltpu.async_copy(rows.at[b], acc.at[ib.at[q].at[1]], ss[b],
                             add=True)

            @pl.when(j >= 1)
            def _():
                pltpu.make_async_copy(rows.at[nb], acc.at[pl.ds(0, CHUNK)],
                                      ss[nb]).wait()

            @pl.when(j + 1 < CPW)
            def _():
                pltpu.make_async_copy(sd_hbm.at[0], ib.at[q1], si[q1]).wait()
                pltpu.async_copy(h_hbm.at[ib.at[q1].at[0]], rows.at[nb],
                                 sg[nb])

            @pl.when(j + 2 < CPW)
            def _():
                pltpu.async_copy(sd_hbm.at[base + j + 2], ib.at[q2], si[q2])

        def step(i, _):
            j = 4 * i
            quarter(j, 0, 0)
            quarter(j + 1, 1, 1)
            quarter(j + 2, 0, 2)
            quarter(j + 3, 1, 3)
            return 0

        lax.fori_loop(0, CPW // 4, step, 0)
        pltpu.make_async_copy(rows.at[1], acc.at[pl.ds(0, CHUNK)],
                              ss[1]).wait()
        plsc.subcore_barrier()
        pltpu.sync_copy(acc.at[pl.ds(t * TROWS, TROWS)],
                        out_hbm.at[c].at[pl.ds(t * TROWS, TROWS)])

    return k


def _sc_degree():
    @functools.partial(
        pl.kernel,
        out_type=jax.ShapeDtypeStruct((NC, NP, 128), jnp.float32),
        mesh=plsc.VectorSubcoreMesh(**_MESH),
        scratch_types=[
            pltpu.VMEM((4, 2, CHUNK), jnp.int32),
            pltpu.VMEM((CHUNK, 128), jnp.float32),
            pltpu.VMEM((ZCH, 128), jnp.float32),
            pltpu.VMEM_SHARED((NP, 128), jnp.float32),
        ] + [pltpu.SemaphoreType.DMA] * 8,
    )
    def k(sd_hbm, ones_hbm, zeros_hbm, out_hbm, ib, onesv, zb, acc,
          si0, si1, si2, si3, ss0, ss1, ss2, ss3):
        si = (si0, si1, si2, si3)
        ss = (ss0, ss1, ss2, ss3)
        c = lax.axis_index("c")
        t = lax.axis_index("s")
        wid = c * NS + t
        base = wid * CPW
        pltpu.sync_copy(zeros_hbm, zb)
        for z in range(TROWS // ZCH):
            pltpu.sync_copy(zb, acc.at[pl.ds(t * TROWS + z * ZCH, ZCH)])
        pltpu.sync_copy(ones_hbm, onesv)
        pltpu.async_copy(sd_hbm.at[base], ib.at[0], si0)
        pltpu.async_copy(sd_hbm.at[base + 1], ib.at[1], si1)
        plsc.subcore_barrier()

        def quarter(j, s):
            s2 = (s + 2) % 4
            pltpu.make_async_copy(sd_hbm.at[0], ib.at[s], si[s]).wait()
            pltpu.async_copy(onesv, acc.at[ib.at[s].at[1]], ss[s], add=True)

            @pl.when(j + 2 < CPW)
            def _():
                @pl.when(j >= 2)
                def _():
                    pltpu.make_async_copy(onesv, acc.at[pl.ds(0, CHUNK)],
                                          ss[s2]).wait()
                pltpu.async_copy(sd_hbm.at[base + j + 2], ib.at[s2], si[s2])

        def step(i, _):
            j = 4 * i
            quarter(j, 0)
            quarter(j + 1, 1)
            quarter(j + 2, 2)
            quarter(j + 3, 3)
            return 0

        lax.fori_loop(0, CPW // 4, step, 0)
        for s in range(4):
            pltpu.make_async_copy(onesv, acc.at[pl.ds(0, CHUNK)],
                                  ss[s]).wait()
        plsc.subcore_barrier()
        pltpu.sync_copy(acc.at[pl.ds(t * TROWS, TROWS)],
                        out_hbm.at[c].at[pl.ds(t * TROWS, TROWS)])

    return k


_ROWS_B = 1024
_GRID = NP // _ROWS_B


def _dinv(deg_ref):
    return lax.rsqrt(deg_ref[0, :, 0:1] + deg_ref[1, :, 0:1] + 1.0)


def _tc1_body(x_ref, w_ref, deg_ref, o_ref):
    dinv = _dinv(deg_ref)
    o_ref[...] = jnp.dot(x_ref[...], w_ref[...],
                         preferred_element_type=jnp.float32) * dinv


def _tc2_body(a_ref, h_ref, deg_ref, b_ref, w_ref, o_ref):
    dinv = _dinv(deg_ref)
    z = jnp.maximum(dinv * (a_ref[0] + a_ref[1] + h_ref[...]) + b_ref[...],
                    0.0)
    o_ref[...] = jnp.dot(z, w_ref[...],
                         preferred_element_type=jnp.float32) * dinv


def _tc3_body(a_ref, h_ref, deg_ref, b_ref, o_ref):
    dinv = _dinv(deg_ref)
    agg = (a_ref[0] + a_ref[1] + h_ref[...])[:, :64]
    o_ref[...] = jnp.maximum(dinv * agg + b_ref[...], 0.0)


def _rows_spec(fw):
    return pl.BlockSpec((_ROWS_B, fw), lambda i: (i, 0))


def _part_spec(fw):
    return pl.BlockSpec((NC, _ROWS_B, fw), lambda i: (0, i, 0))


def _full_spec(a, b):
    return pl.BlockSpec((a, b), lambda i: (0, 0))


def kernel(x, edge_index, W1, b1, W2, b2):
    s = edge_index[0].astype(jnp.int32)
    d = edge_index[1].astype(jnp.int32)
    pad = EP - E
    padrows = N + (jnp.arange(pad, dtype=jnp.int32) % (NP - N))
    s2 = jnp.concatenate([s, padrows]).reshape(NCHUNKS, CHUNK)
    d2 = jnp.concatenate([d, padrows]).reshape(NCHUNKS, CHUNK)
    sd2 = jnp.stack([s2, d2], axis=1)
    xp = jnp.pad(x, ((0, NP - N), (0, 0)))
    W2p = jnp.pad(W2, ((0, 0), (0, 64)))
    ones128 = jnp.ones((CHUNK, 128), jnp.float32)
    z128 = jnp.zeros((ZCH, 128), jnp.float32)

    degp = _sc_degree()(sd2, ones128, z128)

    h1s = pl.pallas_call(
        _tc1_body,
        grid=(_GRID,),
        in_specs=[_rows_spec(128), _full_spec(128, 128), _part_spec(128)],
        out_specs=_rows_spec(128),
        out_shape=jax.ShapeDtypeStruct((NP, 128), jnp.float32),
    )(xp, W1, degp)

    sc_scatter = _sc_scatter()
    a1 = sc_scatter(h1s, sd2, z128)

    h2s = pl.pallas_call(
        _tc2_body,
        grid=(_GRID,),
        in_specs=[_part_spec(128), _rows_spec(128), _part_spec(128),
                  _full_spec(1, 128), _full_spec(128, 128)],
        out_specs=_rows_spec(128),
        out_shape=jax.ShapeDtypeStruct((NP, 128), jnp.float32),
    )(a1, h1s, degp, b1.reshape(1, 128), W2p)

    a2 = sc_scatter(h2s, sd2, z128)

    outp = pl.pallas_call(
        _tc3_body,
        grid=(_GRID,),
        in_specs=[_part_spec(128), _rows_spec(128), _part_spec(128),
                  _full_spec(1, 64)],
        out_specs=_rows_spec(64),
        out_shape=jax.ShapeDtypeStruct((NP, 64), jnp.float32),
    )(a2, h2s, degp, b2.reshape(1, 64))

    return outp[:N]


# speedup vs baseline: 1.0227x; 1.0227x over previous
import functools

import jax
import jax.numpy as jnp
from jax import lax
from jax.experimental import pallas as pl
from jax.experimental.pallas import tpu as pltpu
from jax.experimental.pallas import tpu_sc as plsc

N = 10000
E = 320000
NP = 10240
EP = 327680
CHUNK = 128
NCHUNKS = EP // CHUNK
NC, NS = 2, 16
NW = NC * NS
CPW = NCHUNKS // NW
TROWS = NP // NS
ZCH = 128

_MESH = dict(core_axis_name="c", subcore_axis_name="s", num_cores=NC,
             num_subcores=NS)


def _sc_scatter():
    @functools.partial(
        pl.kernel,
        out_type=jax.ShapeDtypeStruct((NC, NP, 128), jnp.float32),
        mesh=plsc.VectorSubcoreMesh(**_MESH),
        scratch_types=[
            pltpu.VMEM((4, 2, CHUNK), jnp.int32),
            pltpu.VMEM((2, CHUNK, 128), jnp.float32),
            pltpu.VMEM_SHARED((NP, 128), jnp.float32),
        ] + [pltpu.SemaphoreType.DMA] * 8,
    )
    def k(h_hbm, sd_hbm, zeros_hbm, out_hbm, ib, rows, acc,
          si0, si1, si2, si3, sg0, sg1, ss0, ss1):
        si = (si0, si1, si2, si3)
        sg = (sg0, sg1)
        ss = (ss0, ss1)
        c = lax.axis_index("c")
        t = lax.axis_index("s")
        wid = c * NS + t
        base = wid * CPW
        pltpu.sync_copy(zeros_hbm, rows.at[0].at[pl.ds(0, ZCH)])
        for z in range(TROWS // ZCH):
            pltpu.sync_copy(rows.at[0].at[pl.ds(0, ZCH)],
                            acc.at[pl.ds(t * TROWS + z * ZCH, ZCH)])
        pltpu.async_copy(sd_hbm.at[base], ib.at[0], si0)
        pltpu.async_copy(sd_hbm.at[base + 1], ib.at[1], si1)
        pltpu.make_async_copy(sd_hbm.at[0], ib.at[0], si0).wait()
        pltpu.async_copy(h_hbm.at[ib.at[0].at[0]], rows.at[0], sg0)
        plsc.subcore_barrier()

        def quarter(j, b, q):
            nb, q1, q2 = 1 - b, (q + 1) % 4, (q + 2) % 4

            @pl.when(j + 1 < CPW)
            def _():
                pltpu.make_async_copy(sd_hbm.at[0], ib.at[q1], si[q1]).wait()
                pltpu.async_copy(h_hbm.at[ib.at[q1].at[0]], rows.at[nb],
                                 sg[nb])

            pltpu.make_async_copy(h_hbm.at[pl.ds(0, CHUNK)], rows.at[b],
                                  sg[b]).wait()
            pltpu.sync_copy(rows.at[b], acc.at[ib.at[q].at[1]], add=True)

            @pl.when(j + 2 < CPW)
            def _():
                pltpu.async_copy(sd_hbm.at[base + j + 2], ib.at[q2], si[q2])

        def step(i, _):
            j = 4 * i
            quarter(j, 0, 0)
            quarter(j + 1, 1, 1)
            quarter(j + 2, 0, 2)
            quarter(j + 3, 1, 3)
            return 0

        lax.fori_loop(0, CPW // 4, step, 0)
        plsc.subcore_barrier()
        pltpu.sync_copy(acc.at[pl.ds(t * TROWS, TROWS)],
                        out_hbm.at[c].at[pl.ds(t * TROWS, TROWS)])

    return k


def _sc_degree():
    @functools.partial(
        pl.kernel,
        out_type=jax.ShapeDtypeStruct((NC, NP, 128), jnp.float32),
        mesh=plsc.VectorSubcoreMesh(**_MESH),
        scratch_types=[
            pltpu.VMEM((4, 2, CHUNK), jnp.int32),
            pltpu.VMEM((CHUNK, 128), jnp.float32),
            pltpu.VMEM((ZCH, 128), jnp.float32),
            pltpu.VMEM_SHARED((NP, 128), jnp.float32),
        ] + [pltpu.SemaphoreType.DMA] * 8,
    )
    def k(sd_hbm, ones_hbm, zeros_hbm, out_hbm, ib, onesv, zb, acc,
          si0, si1, si2, si3, ss0, ss1, ss2, ss3):
        si = (si0, si1, si2, si3)
        ss = (ss0, ss1, ss2, ss3)
        c = lax.axis_index("c")
        t = lax.axis_index("s")
        wid = c * NS + t
        base = wid * CPW
        pltpu.sync_copy(zeros_hbm, zb)
        for z in range(TROWS // ZCH):
            pltpu.sync_copy(zb, acc.at[pl.ds(t * TROWS + z * ZCH, ZCH)])
        pltpu.sync_copy(ones_hbm, onesv)
        pltpu.async_copy(sd_hbm.at[base], ib.at[0], si0)
        pltpu.async_copy(sd_hbm.at[base + 1], ib.at[1], si1)
        plsc.subcore_barrier()

        def quarter(j, s):
            s2 = (s + 2) % 4
            pltpu.make_async_copy(sd_hbm.at[0], ib.at[s], si[s]).wait()
            pltpu.async_copy(onesv, acc.at[ib.at[s].at[1]], ss[s], add=True)

            @pl.when(j + 2 < CPW)
            def _():
                @pl.when(j >= 2)
                def _():
                    pltpu.make_async_copy(onesv, acc.at[pl.ds(0, CHUNK)],
                                          ss[s2]).wait()
                pltpu.async_copy(sd_hbm.at[base + j + 2], ib.at[s2], si[s2])

        def step(i, _):
            j = 4 * i
            quarter(j, 0)
            quarter(j + 1, 1)
            quarter(j + 2, 2)
            quarter(j + 3, 3)
            return 0

        lax.fori_loop(0, CPW // 4, step, 0)
        for s in range(4):
            pltpu.make_async_copy(onesv, acc.at[pl.ds(0, CHUNK)],
                                  ss[s]).wait()
        plsc.subcore_barrier()
        pltpu.sync_copy(acc.at[pl.ds(t * TROWS, TROWS)],
                        out_hbm.at[c].at[pl.ds(t * TROWS, TROWS)])

    return k


_ROWS_B = 1024
_GRID = NP // _ROWS_B


def _tc0_body(x_ref, w_ref, o_ref):
    o_ref[...] = jnp.dot(x_ref[...], w_ref[...],
                         preferred_element_type=jnp.float32)


def _tc1_body(xw_ref, deg_ref, o_ref, dinv_ref):
    dinv = lax.rsqrt(deg_ref[0, :, 0:1] + deg_ref[1, :, 0:1] + 1.0)
    o_ref[...] = xw_ref[...] * dinv
    dinv_ref[...] = dinv


def _tc2_body(a_ref, h_ref, dinv_ref, b_ref, w_ref, o_ref):
    dinv = dinv_ref[...]
    z = jnp.maximum(dinv * (a_ref[0] + a_ref[1] + h_ref[...]) + b_ref[...],
                    0.0)
    o_ref[...] = jnp.dot(z, w_ref[...],
                         preferred_element_type=jnp.float32) * dinv


def _tc3_body(a_ref, h_ref, dinv_ref, b_ref, o_ref):
    dinv = dinv_ref[...]
    agg = (a_ref[0] + a_ref[1] + h_ref[...])[:, :64]
    o_ref[...] = jnp.maximum(dinv * agg + b_ref[...], 0.0)


def _rows_spec(fw):
    return pl.BlockSpec((_ROWS_B, fw), lambda i: (i, 0))


def _part_spec(fw):
    return pl.BlockSpec((NC, _ROWS_B, fw), lambda i: (0, i, 0))


def _full_spec(a, b):
    return pl.BlockSpec((a, b), lambda i: (0, 0))


def kernel(x, edge_index, W1, b1, W2, b2):
    s = edge_index[0].astype(jnp.int32)
    d = edge_index[1].astype(jnp.int32)
    pad = EP - E
    padrows = N + (jnp.arange(pad, dtype=jnp.int32) % (NP - N))
    s2 = jnp.concatenate([s, padrows]).reshape(NCHUNKS, CHUNK)
    d2 = jnp.concatenate([d, padrows]).reshape(NCHUNKS, CHUNK)
    sd2 = jnp.stack([s2, d2], axis=1)
    xp = jnp.pad(x, ((0, NP - N), (0, 0)))
    W2p = jnp.pad(W2, ((0, 0), (0, 64)))
    ones128 = jnp.ones((CHUNK, 128), jnp.float32)
    z128 = jnp.zeros((ZCH, 128), jnp.float32)

    degp = _sc_degree()(sd2, ones128, z128)
    xw1 = pl.pallas_call(
        _tc0_body,
        grid=(_GRID,),
        in_specs=[_rows_spec(128), _full_spec(128, 128)],
        out_specs=_rows_spec(128),
        out_shape=jax.ShapeDtypeStruct((NP, 128), jnp.float32),
    )(xp, W1)

    h1s, dinv = pl.pallas_call(
        _tc1_body,
        grid=(_GRID,),
        in_specs=[_rows_spec(128), _part_spec(128)],
        out_specs=(_rows_spec(128), _rows_spec(1)),
        out_shape=(jax.ShapeDtypeStruct((NP, 128), jnp.float32),
                   jax.ShapeDtypeStruct((NP, 1), jnp.float32)),
    )(xw1, degp)

    sc_scatter = _sc_scatter()
    a1 = sc_scatter(h1s, sd2, z128)

    h2s = pl.pallas_call(
        _tc2_body,
        grid=(_GRID,),
        in_specs=[_part_spec(128), _rows_spec(128), _rows_spec(1),
                  _full_spec(1, 128), _full_spec(128, 128)],
        out_specs=_rows_spec(128),
        out_shape=jax.ShapeDtypeStruct((NP, 128), jnp.float32),
    )(a1, h1s, dinv, b1.reshape(1, 128), W2p)

    a2 = sc_scatter(h2s, sd2, z128)

    outp = pl.pallas_call(
        _tc3_body,
        grid=(_GRID,),
        in_specs=[_part_spec(128), _rows_spec(128), _rows_spec(1),
                  _full_spec(1, 64)],
        out_specs=_rows_spec(64),
        out_shape=jax.ShapeDtypeStruct((NP, 64), jnp.float32),
    )(a2, h2s, dinv, b2.reshape(1, 64))

    return outp[:N]


# speedup vs baseline: 1.2042x; 1.1775x over previous
import functools

import jax
import jax.numpy as jnp
from jax import lax
from jax.experimental import pallas as pl
from jax.experimental.pallas import tpu as pltpu
from jax.experimental.pallas import tpu_sc as plsc

N = 10000
E = 320000
NP = 10240
EP = 327680
CHUNK = 128
NCHUNKS = EP // CHUNK
NC, NS = 2, 16
NW = NC * NS
CPW = NCHUNKS // NW
TROWS = NP // NS
ZCH = 128
_EPT = EP // NW
_VL = 16

_MESH = dict(core_axis_name="c", subcore_axis_name="s", num_cores=NC,
             num_subcores=NS)


def _sc_scatter():
    @functools.partial(
        pl.kernel,
        out_type=jax.ShapeDtypeStruct((NC, NP, 128), jnp.float32),
        mesh=plsc.VectorSubcoreMesh(**_MESH),
        scratch_types=[
            pltpu.VMEM((4, 2, CHUNK), jnp.int32),
            pltpu.VMEM((2, CHUNK, 128), jnp.float32),
            pltpu.VMEM_SHARED((NP, 128), jnp.float32),
        ] + [pltpu.SemaphoreType.DMA] * 8,
    )
    def k(h_hbm, sd_hbm, zeros_hbm, out_hbm, ib, rows, acc,
          si0, si1, si2, si3, sg0, sg1, ss0, ss1):
        si = (si0, si1, si2, si3)
        sg = (sg0, sg1)
        ss = (ss0, ss1)
        c = lax.axis_index("c")
        t = lax.axis_index("s")
        wid = c * NS + t
        base = wid * CPW
        pltpu.sync_copy(zeros_hbm, rows.at[0].at[pl.ds(0, ZCH)])
        for z in range(TROWS // ZCH):
            pltpu.sync_copy(rows.at[0].at[pl.ds(0, ZCH)],
                            acc.at[pl.ds(t * TROWS + z * ZCH, ZCH)])
        pltpu.async_copy(sd_hbm.at[base], ib.at[0], si0)
        pltpu.async_copy(sd_hbm.at[base + 1], ib.at[1], si1)
        pltpu.make_async_copy(sd_hbm.at[0], ib.at[0], si0).wait()
        pltpu.async_copy(h_hbm.at[ib.at[0].at[0]], rows.at[0], sg0)
        plsc.subcore_barrier()

        def quarter(j, b, q):
            nb, q1, q2 = 1 - b, (q + 1) % 4, (q + 2) % 4

            @pl.when(j + 1 < CPW)
            def _():
                pltpu.make_async_copy(sd_hbm.at[0], ib.at[q1], si[q1]).wait()
                pltpu.async_copy(h_hbm.at[ib.at[q1].at[0]], rows.at[nb],
                                 sg[nb])

            pltpu.make_async_copy(h_hbm.at[pl.ds(0, CHUNK)], rows.at[b],
                                  sg[b]).wait()
            pltpu.sync_copy(rows.at[b], acc.at[ib.at[q].at[1]], add=True)

            @pl.when(j + 2 < CPW)
            def _():
                pltpu.async_copy(sd_hbm.at[base + j + 2], ib.at[q2], si[q2])

        def step(i, _):
            j = 4 * i
            quarter(j, 0, 0)
            quarter(j + 1, 1, 1)
            quarter(j + 2, 0, 2)
            quarter(j + 3, 1, 3)
            return 0

        lax.fori_loop(0, CPW // 4, step, 0)
        plsc.subcore_barrier()
        pltpu.sync_copy(acc.at[pl.ds(t * TROWS, TROWS)],
                        out_hbm.at[c].at[pl.ds(t * TROWS, TROWS)])

    return k


def _sc_degree():
    @functools.partial(
        pl.kernel,
        out_type=jax.ShapeDtypeStruct((NC, NP, 128), jnp.float32),
        mesh=plsc.VectorSubcoreMesh(**_MESH),
        compiler_params=pltpu.CompilerParams(needs_layout_passes=False),
        scratch_types=[
            pltpu.VMEM((_EPT,), jnp.int32),
            pltpu.VMEM((NP,), jnp.float32),
            pltpu.VMEM((NS, TROWS), jnp.float32),
            pltpu.VMEM((ZCH, 128), jnp.float32),
            pltpu.VMEM_SHARED((NS, NP), jnp.float32),
        ],
    )
    def k(d_hbm, out_hbm, dv, hist, slab, colbuf, sh):
        c = lax.axis_index("c")
        t = lax.axis_index("s")
        wid = c * NS + t
        pltpu.sync_copy(d_hbm.at[wid], dv)
        zeros16 = jnp.zeros((_VL,), jnp.float32)
        ones16 = jnp.ones((_VL,), jnp.float32)

        def zero_step(i, _):
            hist[pl.ds(i * _VL, _VL)] = zeros16
            return 0

        lax.fori_loop(0, NP // _VL, zero_step, 0)

        def hist_step(i, _):
            idx = dv[pl.ds(i * _VL, _VL)]
            plsc.addupdate_scatter(hist, [idx], ones16)
            return 0

        lax.fori_loop(0, _EPT // _VL, hist_step, 0)
        pltpu.sync_copy(hist, sh.at[t])
        plsc.subcore_barrier()
        for kk in range(NS):
            pltpu.sync_copy(sh.at[kk].at[pl.ds(t * TROWS, TROWS)],
                            slab.at[kk])
        col0 = jnp.zeros((_VL,), jnp.int32)
        iota16 = lax.iota(jnp.int32, _VL)

        def pub_step(z, _):
            for j in range(ZCH // _VL):
                deg16 = zeros16
                for kk in range(NS):
                    deg16 = deg16 + slab[kk, pl.ds(z * ZCH + j * _VL, _VL)]
                plsc.store_scatter(colbuf, [j * _VL + iota16, col0], deg16)
            pltpu.sync_copy(colbuf,
                            out_hbm.at[c].at[pl.ds(t * TROWS + z * ZCH,
                                                   ZCH)])
            return 0

        lax.fori_loop(0, TROWS // ZCH, pub_step, 0)

    return k


_ROWS_B = 1024
_GRID = NP // _ROWS_B


def _tc0_body(x_ref, w_ref, o_ref):
    o_ref[...] = jnp.dot(x_ref[...], w_ref[...],
                         preferred_element_type=jnp.float32)


def _tc1_body(xw_ref, deg_ref, o_ref, dinv_ref):
    dinv = lax.rsqrt(deg_ref[0, :, 0:1] + deg_ref[1, :, 0:1] + 1.0)
    o_ref[...] = xw_ref[...] * dinv
    dinv_ref[...] = dinv


def _tc2_body(a_ref, h_ref, dinv_ref, b_ref, w_ref, o_ref):
    dinv = dinv_ref[...]
    z = jnp.maximum(dinv * (a_ref[0] + a_ref[1] + h_ref[...]) + b_ref[...],
                    0.0)
    o_ref[...] = jnp.dot(z, w_ref[...],
                         preferred_element_type=jnp.float32) * dinv


def _tc3_body(a_ref, h_ref, dinv_ref, b_ref, o_ref):
    dinv = dinv_ref[...]
    agg = (a_ref[0] + a_ref[1] + h_ref[...])[:, :64]
    o_ref[...] = jnp.maximum(dinv * agg + b_ref[...], 0.0)


def _rows_spec(fw):
    return pl.BlockSpec((_ROWS_B, fw), lambda i: (i, 0))


def _part_spec(fw):
    return pl.BlockSpec((NC, _ROWS_B, fw), lambda i: (0, i, 0))


def _full_spec(a, b):
    return pl.BlockSpec((a, b), lambda i: (0, 0))


def kernel(x, edge_index, W1, b1, W2, b2):
    s = edge_index[0].astype(jnp.int32)
    d = edge_index[1].astype(jnp.int32)
    pad = EP - E
    padrows = N + (jnp.arange(pad, dtype=jnp.int32) % (NP - N))
    s2 = jnp.concatenate([s, padrows]).reshape(NCHUNKS, CHUNK)
    d2 = jnp.concatenate([d, padrows]).reshape(NCHUNKS, CHUNK)
    sd2 = jnp.stack([s2, d2], axis=1)
    xp = jnp.pad(x, ((0, NP - N), (0, 0)))
    W2p = jnp.pad(W2, ((0, 0), (0, 64)))
    z128 = jnp.zeros((ZCH, 128), jnp.float32)

    degp = _sc_degree()(d2.reshape(NW, _EPT))
    xw1 = pl.pallas_call(
        _tc0_body,
        grid=(_GRID,),
        in_specs=[_rows_spec(128), _full_spec(128, 128)],
        out_specs=_rows_spec(128),
        out_shape=jax.ShapeDtypeStruct((NP, 128), jnp.float32),
    )(xp, W1)

    h1s, dinv = pl.pallas_call(
        _tc1_body,
        grid=(_GRID,),
        in_specs=[_rows_spec(128), _part_spec(128)],
        out_specs=(_rows_spec(128), _rows_spec(1)),
        out_shape=(jax.ShapeDtypeStruct((NP, 128), jnp.float32),
                   jax.ShapeDtypeStruct((NP, 1), jnp.float32)),
    )(xw1, degp)

    sc_scatter = _sc_scatter()
    a1 = sc_scatter(h1s, sd2, z128)

    h2s = pl.pallas_call(
        _tc2_body,
        grid=(_GRID,),
        in_specs=[_part_spec(128), _rows_spec(128), _rows_spec(1),
                  _full_spec(1, 128), _full_spec(128, 128)],
        out_specs=_rows_spec(128),
        out_shape=jax.ShapeDtypeStruct((NP, 128), jnp.float32),
    )(a1, h1s, dinv, b1.reshape(1, 128), W2p)

    a2 = sc_scatter(h2s, sd2, z128)

    outp = pl.pallas_call(
        _tc3_body,
        grid=(_GRID,),
        in_specs=[_part_spec(128), _rows_spec(128), _rows_spec(1),
                  _full_spec(1, 64)],
        out_specs=_rows_spec(64),
        out_shape=jax.ShapeDtypeStruct((NP, 64), jnp.float32),
    )(a2, h2s, dinv, b2.reshape(1, 64))

    return outp[:N]


# speedup vs baseline: 1.2097x; 1.0046x over previous
import functools

import jax
import jax.numpy as jnp
from jax import lax
from jax.experimental import pallas as pl
from jax.experimental.pallas import tpu as pltpu
from jax.experimental.pallas import tpu_sc as plsc

N = 10000
E = 320000
NP = 10240
EP = 327680
CHUNK = 128
NCHUNKS = EP // CHUNK
NC, NS = 2, 16
NW = NC * NS
CPW = NCHUNKS // NW
TROWS = NP // NS
ZCH = 128
_EPT = EP // NW
_VL = 16

_MESH = dict(core_axis_name="c", subcore_axis_name="s", num_cores=NC,
             num_subcores=NS)


def _sc_scatter():
    @functools.partial(
        pl.kernel,
        out_type=jax.ShapeDtypeStruct((NC, NP, 128), jnp.float32),
        mesh=plsc.VectorSubcoreMesh(**_MESH),
        scratch_types=[
            pltpu.VMEM((4, 2, CHUNK), jnp.int32),
            pltpu.VMEM((2, CHUNK, 128), jnp.float32),
            pltpu.VMEM_SHARED((NP, 128), jnp.float32),
        ] + [pltpu.SemaphoreType.DMA] * 8,
    )
    def k(h_hbm, sd_hbm, zeros_hbm, out_hbm, ib, rows, acc,
          si0, si1, si2, si3, sg0, sg1, ss0, ss1):
        si = (si0, si1, si2, si3)
        sg = (sg0, sg1)
        ss = (ss0, ss1)
        c = lax.axis_index("c")
        t = lax.axis_index("s")
        wid = c * NS + t
        base = wid * CPW
        pltpu.sync_copy(zeros_hbm, rows.at[0].at[pl.ds(0, ZCH)])
        for z in range(TROWS // ZCH):
            pltpu.sync_copy(rows.at[0].at[pl.ds(0, ZCH)],
                            acc.at[pl.ds(t * TROWS + z * ZCH, ZCH)])
        pltpu.async_copy(sd_hbm.at[base], ib.at[0], si0)
        pltpu.async_copy(sd_hbm.at[base + 1], ib.at[1], si1)
        pltpu.make_async_copy(sd_hbm.at[0], ib.at[0], si0).wait()
        pltpu.async_copy(h_hbm.at[ib.at[0].at[0]], rows.at[0], sg0)
        plsc.subcore_barrier()

        def quarter(j, b, q):
            nb, q1, q2 = 1 - b, (q + 1) % 4, (q + 2) % 4

            @pl.when(j + 1 < CPW)
            def _():
                pltpu.make_async_copy(sd_hbm.at[0], ib.at[q1], si[q1]).wait()
                pltpu.async_copy(h_hbm.at[ib.at[q1].at[0]], rows.at[nb],
                                 sg[nb])

            pltpu.make_async_copy(h_hbm.at[pl.ds(0, CHUNK)], rows.at[b],
                                  sg[b]).wait()
            pltpu.sync_copy(rows.at[b], acc.at[ib.at[q].at[1]], add=True)

            @pl.when(j + 2 < CPW)
            def _():
                pltpu.async_copy(sd_hbm.at[base + j + 2], ib.at[q2], si[q2])

        def step(i, _):
            j = 4 * i
            quarter(j, 0, 0)
            quarter(j + 1, 1, 1)
            quarter(j + 2, 0, 2)
            quarter(j + 3, 1, 3)
            return 0

        lax.fori_loop(0, CPW // 4, step, 0)
        plsc.subcore_barrier()
        pltpu.sync_copy(acc.at[pl.ds(t * TROWS, TROWS)],
                        out_hbm.at[c].at[pl.ds(t * TROWS, TROWS)])

    return k


def _sc_degree():
    @functools.partial(
        pl.kernel,
        out_type=jax.ShapeDtypeStruct((NC, NP, 128), jnp.float32),
        mesh=plsc.VectorSubcoreMesh(**_MESH),
        compiler_params=pltpu.CompilerParams(needs_layout_passes=False),
        scratch_types=[
            pltpu.VMEM((_EPT,), jnp.int32),
            pltpu.VMEM((NP,), jnp.float32),
            pltpu.VMEM((NS, TROWS), jnp.float32),
            pltpu.VMEM((ZCH, 128), jnp.float32),
            pltpu.VMEM_SHARED((NS, NP), jnp.float32),
        ],
    )
    def k(d_hbm, out_hbm, dv, hist, slab, colbuf, sh):
        c = lax.axis_index("c")
        t = lax.axis_index("s")
        wid = c * NS + t
        pltpu.sync_copy(d_hbm.at[wid], dv)
        zeros16 = jnp.zeros((_VL,), jnp.float32)
        ones16 = jnp.ones((_VL,), jnp.float32)

        def zero_step(i, _):
            hist[pl.ds(i * _VL, _VL)] = zeros16
            return 0

        lax.fori_loop(0, NP // _VL, zero_step, 0)

        def hist_step(i, _):
            idx = dv[pl.ds(i * _VL, _VL)]
            plsc.addupdate_scatter(hist, [idx], ones16)
            return 0

        lax.fori_loop(0, _EPT // _VL, hist_step, 0)
        pltpu.sync_copy(hist, sh.at[t])
        plsc.subcore_barrier()
        for kk in range(NS):
            pltpu.sync_copy(sh.at[kk].at[pl.ds(t * TROWS, TROWS)],
                            slab.at[kk])
        col0 = jnp.zeros((_VL,), jnp.int32)
        iota16 = lax.iota(jnp.int32, _VL)

        def pub_step(z, _):
            for j in range(ZCH // _VL):
                deg16 = zeros16
                for kk in range(NS):
                    deg16 = deg16 + slab[kk, pl.ds(z * ZCH + j * _VL, _VL)]
                plsc.store_scatter(colbuf, [j * _VL + iota16, col0], deg16)
            pltpu.sync_copy(colbuf,
                            out_hbm.at[c].at[pl.ds(t * TROWS + z * ZCH,
                                                   ZCH)])
            return 0

        lax.fori_loop(0, TROWS // ZCH, pub_step, 0)

    return k


_ROWS_B = 1024
_GRID = NP // _ROWS_B


def _tc1_body(x_ref, w_ref, deg_ref, o_ref, dinv_ref):
    dinv = lax.rsqrt(deg_ref[0, :, 0:1] + deg_ref[1, :, 0:1] + 1.0)
    o_ref[...] = jnp.dot(x_ref[...], w_ref[...],
                         preferred_element_type=jnp.float32) * dinv
    dinv_ref[...] = dinv


def _tc2_body(a_ref, h_ref, dinv_ref, b_ref, w_ref, o_ref):
    dinv = dinv_ref[...]
    z = jnp.maximum(dinv * (a_ref[0] + a_ref[1] + h_ref[...]) + b_ref[...],
                    0.0)
    o_ref[...] = jnp.dot(z, w_ref[...],
                         preferred_element_type=jnp.float32) * dinv


def _tc3_body(a_ref, h_ref, dinv_ref, b_ref, o_ref):
    dinv = dinv_ref[...]
    agg = (a_ref[0] + a_ref[1] + h_ref[...])[:, :64]
    o_ref[...] = jnp.maximum(dinv * agg + b_ref[...], 0.0)


def _rows_spec(fw):
    return pl.BlockSpec((_ROWS_B, fw), lambda i: (i, 0))


def _part_spec(fw):
    return pl.BlockSpec((NC, _ROWS_B, fw), lambda i: (0, i, 0))


def _full_spec(a, b):
    return pl.BlockSpec((a, b), lambda i: (0, 0))


def kernel(x, edge_index, W1, b1, W2, b2):
    s = edge_index[0].astype(jnp.int32)
    d = edge_index[1].astype(jnp.int32)
    pad = EP - E
    padrows = N + (jnp.arange(pad, dtype=jnp.int32) % (NP - N))
    s2 = jnp.concatenate([s, padrows]).reshape(NCHUNKS, CHUNK)
    d2 = jnp.concatenate([d, padrows]).reshape(NCHUNKS, CHUNK)
    sd2 = jnp.stack([s2, d2], axis=1)
    xp = jnp.pad(x, ((0, NP - N), (0, 0)))
    W2p = jnp.pad(W2, ((0, 0), (0, 64)))
    z128 = jnp.zeros((ZCH, 128), jnp.float32)

    degp = _sc_degree()(d2.reshape(NW, _EPT))

    h1s, dinv = pl.pallas_call(
        _tc1_body,
        grid=(_GRID,),
        in_specs=[_rows_spec(128), _full_spec(128, 128), _part_spec(128)],
        out_specs=(_rows_spec(128), _rows_spec(1)),
        out_shape=(jax.ShapeDtypeStruct((NP, 128), jnp.float32),
                   jax.ShapeDtypeStruct((NP, 1), jnp.float32)),
    )(xp, W1, degp)

    sc_scatter = _sc_scatter()
    a1 = sc_scatter(h1s, sd2, z128)

    h2s = pl.pallas_call(
        _tc2_body,
        grid=(_GRID,),
        in_specs=[_part_spec(128), _rows_spec(128), _rows_spec(1),
                  _full_spec(1, 128), _full_spec(128, 128)],
        out_specs=_rows_spec(128),
        out_shape=jax.ShapeDtypeStruct((NP, 128), jnp.float32),
    )(a1, h1s, dinv, b1.reshape(1, 128), W2p)

    a2 = sc_scatter(h2s, sd2, z128)

    outp = pl.pallas_call(
        _tc3_body,
        grid=(_GRID,),
        in_specs=[_part_spec(128), _rows_spec(128), _rows_spec(1),
                  _full_spec(1, 64)],
        out_specs=_rows_spec(64),
        out_shape=jax.ShapeDtypeStruct((NP, 64), jnp.float32),
    )(a2, h2s, dinv, b2.reshape(1, 64))

    return outp[:N]


# speedup vs baseline: 1.2222x; 1.0103x over previous
import functools

import jax
import jax.numpy as jnp
from jax import lax
from jax.experimental import pallas as pl
from jax.experimental.pallas import tpu as pltpu
from jax.experimental.pallas import tpu_sc as plsc

N = 10000
E = 320000
NP = 10240
EP = 327680
CHUNK = 128
NCHUNKS = EP // CHUNK
NC, NS = 2, 16
NW = NC * NS
CPW = NCHUNKS // NW
TROWS = NP // NS
ZCH = 128
_EPT = EP // NW
_VL = 16

_MESH = dict(core_axis_name="c", subcore_axis_name="s", num_cores=NC,
             num_subcores=NS)


def _sc_scatter():
    @functools.partial(
        pl.kernel,
        out_type=jax.ShapeDtypeStruct((NC, NP, 128), jnp.float32),
        mesh=plsc.VectorSubcoreMesh(**_MESH),
        scratch_types=[
            pltpu.VMEM((4, 2, CHUNK), jnp.int32),
            pltpu.VMEM((2, CHUNK, 128), jnp.float32),
            pltpu.VMEM_SHARED((NP, 128), jnp.float32),
        ] + [pltpu.SemaphoreType.DMA] * 8,
    )
    def k(h_hbm, sd_hbm, zeros_hbm, out_hbm, ib, rows, acc,
          si0, si1, si2, si3, sg0, sg1, ss0, ss1):
        si = (si0, si1, si2, si3)
        sg = (sg0, sg1)
        ss = (ss0, ss1)
        c = lax.axis_index("c")
        t = lax.axis_index("s")
        wid = c * NS + t
        base = wid * CPW
        pltpu.async_copy(sd_hbm.at[base], ib.at[0], si0)
        pltpu.async_copy(sd_hbm.at[base + 1], ib.at[1], si1)
        pltpu.sync_copy(zeros_hbm, rows.at[1].at[pl.ds(0, ZCH)])
        for z in range(TROWS // ZCH):
            pltpu.async_copy(rows.at[1].at[pl.ds(0, ZCH)],
                             acc.at[pl.ds(t * TROWS + z * ZCH, ZCH)], ss0)
        pltpu.make_async_copy(sd_hbm.at[0], ib.at[0], si0).wait()
        pltpu.async_copy(h_hbm.at[ib.at[0].at[0]], rows.at[0], sg0)
        for z in range(TROWS // ZCH):
            pltpu.make_async_copy(rows.at[1].at[pl.ds(0, ZCH)],
                                  acc.at[pl.ds(0, ZCH)], ss0).wait()
        plsc.subcore_barrier()

        def quarter(j, b, q):
            nb, q1, q2 = 1 - b, (q + 1) % 4, (q + 2) % 4

            @pl.when(j + 1 < CPW)
            def _():
                pltpu.make_async_copy(sd_hbm.at[0], ib.at[q1], si[q1]).wait()
                pltpu.async_copy(h_hbm.at[ib.at[q1].at[0]], rows.at[nb],
                                 sg[nb])

            pltpu.make_async_copy(h_hbm.at[pl.ds(0, CHUNK)], rows.at[b],
                                  sg[b]).wait()
            pltpu.sync_copy(rows.at[b], acc.at[ib.at[q].at[1]], add=True)

            @pl.when(j + 2 < CPW)
            def _():
                pltpu.async_copy(sd_hbm.at[base + j + 2], ib.at[q2], si[q2])

        def step(i, _):
            j = 4 * i
            quarter(j, 0, 0)
            quarter(j + 1, 1, 1)
            quarter(j + 2, 0, 2)
            quarter(j + 3, 1, 3)
            return 0

        lax.fori_loop(0, CPW // 4, step, 0)
        plsc.subcore_barrier()
        pltpu.sync_copy(acc.at[pl.ds(t * TROWS, TROWS)],
                        out_hbm.at[c].at[pl.ds(t * TROWS, TROWS)])

    return k


def _sc_degree():
    @functools.partial(
        pl.kernel,
        out_type=jax.ShapeDtypeStruct((NC, NP, 128), jnp.float32),
        mesh=plsc.VectorSubcoreMesh(**_MESH),
        compiler_params=pltpu.CompilerParams(needs_layout_passes=False),
        scratch_types=[
            pltpu.VMEM((_EPT,), jnp.int32),
            pltpu.VMEM((NP,), jnp.float32),
            pltpu.VMEM((NS, TROWS), jnp.float32),
            pltpu.VMEM((ZCH, 128), jnp.float32),
            pltpu.VMEM_SHARED((NS, NP), jnp.float32),
        ],
    )
    def k(d_hbm, out_hbm, dv, hist, slab, colbuf, sh):
        c = lax.axis_index("c")
        t = lax.axis_index("s")
        wid = c * NS + t
        pltpu.sync_copy(d_hbm.at[wid], dv)
        zeros16 = jnp.zeros((_VL,), jnp.float32)
        ones16 = jnp.ones((_VL,), jnp.float32)

        def zero_step(i, _):
            hist[pl.ds(i * _VL, _VL)] = zeros16
            return 0

        lax.fori_loop(0, NP // _VL, zero_step, 0)

        def hist_step(i, _):
            idx = dv[pl.ds(i * _VL, _VL)]
            plsc.addupdate_scatter(hist, [idx], ones16)
            return 0

        lax.fori_loop(0, _EPT // _VL, hist_step, 0)
        pltpu.sync_copy(hist, sh.at[t])
        plsc.subcore_barrier()
        for kk in range(NS):
            pltpu.sync_copy(sh.at[kk].at[pl.ds(t * TROWS, TROWS)],
                            slab.at[kk])
        col0 = jnp.zeros((_VL,), jnp.int32)
        iota16 = lax.iota(jnp.int32, _VL)

        def pub_step(z, _):
            for j in range(ZCH // _VL):
                deg16 = zeros16
                for kk in range(NS):
                    deg16 = deg16 + slab[kk, pl.ds(z * ZCH + j * _VL, _VL)]
                plsc.store_scatter(colbuf, [j * _VL + iota16, col0], deg16)
            pltpu.sync_copy(colbuf,
                            out_hbm.at[c].at[pl.ds(t * TROWS + z * ZCH,
                                                   ZCH)])
            return 0

        lax.fori_loop(0, TROWS // ZCH, pub_step, 0)

    return k


_ROWS_B = 1024
_GRID = NP // _ROWS_B


def _tc1_body(x_ref, w_ref, deg_ref, o_ref, dinv_ref):
    dinv = lax.rsqrt(deg_ref[0, :, 0:1] + deg_ref[1, :, 0:1] + 1.0)
    o_ref[...] = jnp.dot(x_ref[...], w_ref[...],
                         preferred_element_type=jnp.float32) * dinv
    dinv_ref[...] = dinv


def _tc2_body(a_ref, h_ref, dinv_ref, b_ref, w_ref, o_ref):
    dinv = dinv_ref[...]
    z = jnp.maximum(dinv * (a_ref[0] + a_ref[1] + h_ref[...]) + b_ref[...],
                    0.0)
    o_ref[...] = jnp.dot(z, w_ref[...],
                         preferred_element_type=jnp.float32) * dinv


def _tc3_body(a_ref, h_ref, dinv_ref, b_ref, o_ref):
    dinv = dinv_ref[...]
    agg = (a_ref[0] + a_ref[1] + h_ref[...])[:, :64]
    o_ref[...] = jnp.maximum(dinv * agg + b_ref[...], 0.0)


def _rows_spec(fw):
    return pl.BlockSpec((_ROWS_B, fw), lambda i: (i, 0))


def _part_spec(fw):
    return pl.BlockSpec((NC, _ROWS_B, fw), lambda i: (0, i, 0))


def _full_spec(a, b):
    return pl.BlockSpec((a, b), lambda i: (0, 0))


def kernel(x, edge_index, W1, b1, W2, b2):
    s = edge_index[0].astype(jnp.int32)
    d = edge_index[1].astype(jnp.int32)
    pad = EP - E
    padrows = N + (jnp.arange(pad, dtype=jnp.int32) % (NP - N))
    s2 = jnp.concatenate([s, padrows]).reshape(NCHUNKS, CHUNK)
    d2 = jnp.concatenate([d, padrows]).reshape(NCHUNKS, CHUNK)
    sd2 = jnp.stack([s2, d2], axis=1)
    xp = jnp.pad(x, ((0, NP - N), (0, 0)))
    W2p = jnp.pad(W2, ((0, 0), (0, 64)))
    z128 = jnp.zeros((ZCH, 128), jnp.float32)

    degp = _sc_degree()(d2.reshape(NW, _EPT))

    h1s, dinv = pl.pallas_call(
        _tc1_body,
        grid=(_GRID,),
        in_specs=[_rows_spec(128), _full_spec(128, 128), _part_spec(128)],
        out_specs=(_rows_spec(128), _rows_spec(1)),
        out_shape=(jax.ShapeDtypeStruct((NP, 128), jnp.float32),
                   jax.ShapeDtypeStruct((NP, 1), jnp.float32)),
    )(xp, W1, degp)

    sc_scatter = _sc_scatter()
    a1 = sc_scatter(h1s, sd2, z128)

    h2s = pl.pallas_call(
        _tc2_body,
        grid=(_GRID,),
        in_specs=[_part_spec(128), _rows_spec(128), _rows_spec(1),
                  _full_spec(1, 128), _full_spec(128, 128)],
        out_specs=_rows_spec(128),
        out_shape=jax.ShapeDtypeStruct((NP, 128), jnp.float32),
    )(a1, h1s, dinv, b1.reshape(1, 128), W2p)

    a2 = sc_scatter(h2s, sd2, z128)

    outp = pl.pallas_call(
        _tc3_body,
        grid=(_GRID,),
        in_specs=[_part_spec(128), _rows_spec(128), _rows_spec(1),
                  _full_spec(1, 64)],
        out_specs=_rows_spec(64),
        out_shape=jax.ShapeDtypeStruct((NP, 64), jnp.float32),
    )(a2, h2s, dinv, b2.reshape(1, 64))

    return outp[:N]
